# Initial kernel scaffold; baseline (speedup 1.0000x reference)
import jax
import jax.numpy as jnp
from jax.experimental import pallas as pl


def kernel(x, edge_index, edge_attr, h_init_x, h_init_edge_attr, lap_weight, W_e, b_e, W_n, b_n):
    raise NotImplementedError("write your pallas kernel here")


# speedup vs baseline: 3.7829x; 3.7829x over previous
import functools

import jax
import jax.numpy as jnp
from jax import lax
from jax.experimental import pallas as pl
from jax.experimental.pallas import tpu as pltpu
from jax.experimental.pallas import tpu_sc as plsc

N = 10000
E = 160000
D = 128
DE = 16
COEFF = 0.1

NC = 2
NS = 16
NW = NC * NS
EPW = 5120
E_PAD = NW * EPW
IR = E_PAD // 128
IRPW = IR // NW

CRA = 8
NCH_A = IRPW // CRA
CRB = 4
NCH_B = IRPW // CRB

CP = 624
REM = N - NS * CP
DEG_PC = 16000
DEG_PT = DEG_PC // NS

_f32 = jnp.float32
_i32 = jnp.int32


def _edge_body(src2, dst2, q2, psrc, pdst, enew, aggout,
               sidx, didx, qv, gs, gd, ev, agg_s, ps_s, pd_s, sem):
    c = lax.axis_index("c")
    s = lax.axis_index("s")
    wid = s * NC + c

    pltpu.sync_copy(psrc.at[pl.ds(s * CP, CP)], ps_s.at[pl.ds(s * CP, CP)])
    pltpu.sync_copy(pdst.at[pl.ds(s * CP, CP)], pd_s.at[pl.ds(s * CP, CP)])

    def _zb(i, carry):
        ev[i, :] = jnp.zeros((16,), _f32)
        return carry
    lax.fori_loop(0, CP, _zb, 0, unroll=8)
    pltpu.sync_copy(ev.at[pl.ds(0, CP)], agg_s.at[pl.ds(s * CP, CP)])

    @pl.when(s == 0)
    def _zrem():
        pltpu.sync_copy(ev.at[pl.ds(0, REM)], agg_s.at[pl.ds(NS * CP, REM)])
        pltpu.sync_copy(psrc.at[pl.ds(NS * CP, REM)],
                        ps_s.at[pl.ds(NS * CP, REM)])
        pltpu.sync_copy(pdst.at[pl.ds(NS * CP, REM)],
                        pd_s.at[pl.ds(NS * CP, REM)])
    plsc.subcore_barrier()

    for j in range(NCH_A):
        rbase = wid * IRPW + j * CRA
        ebase = rbase * 128
        pltpu.sync_copy(src2.at[pl.ds(rbase, CRA)], sidx)
        pltpu.sync_copy(dst2.at[pl.ds(rbase, CRA)], didx)
        pltpu.sync_copy(q2.at[pl.ds(ebase, CRA * 128)], qv)
        descs = []
        for r in range(CRA):
            descs.append(pltpu.async_copy(
                ps_s.at[sidx.at[r]], gs.at[pl.ds(r * 128, 128)], sem))
            descs.append(pltpu.async_copy(
                pd_s.at[didx.at[r]], gd.at[pl.ds(r * 128, 128)], sem))
        for dsc in descs:
            dsc.wait()

        def _cb(i, carry):
            ev[i, :] = jnp.maximum(qv[i, :] + gs[i, :] + gd[i, :], 0.0)
            return carry
        lax.fori_loop(0, CRA * 128, _cb, 0, unroll=4)

        pltpu.sync_copy(ev, enew.at[pl.ds(ebase, CRA * 128)])
        for r in range(CRA):
            pltpu.sync_copy(ev.at[pl.ds(r * 128, 128)],
                            agg_s.at[didx.at[r]], add=True)

    plsc.subcore_barrier()
    pltpu.sync_copy(agg_s.at[pl.ds(s * CP, CP)],
                    aggout.at[pl.ds(c * N + s * CP, CP)])

    @pl.when(s == 0)
    def _crem():
        pltpu.sync_copy(agg_s.at[pl.ds(NS * CP, REM)],
                        aggout.at[pl.ds(c * N + NS * CP, REM)])


def _edge_sc(src2, dst2, q2, psrc, pdst):
    mesh = plsc.VectorSubcoreMesh(core_axis_name="c", subcore_axis_name="s")
    fn = pl.kernel(
        _edge_body,
        out_type=[jax.ShapeDtypeStruct((E_PAD, DE), _f32),
                  jax.ShapeDtypeStruct((NC * N, DE), _f32)],
        mesh=mesh,
        scratch_types=[
            pltpu.VMEM((CRA, 128), _i32),
            pltpu.VMEM((CRA, 128), _i32),
            pltpu.VMEM((CRA * 128, DE), _f32),
            pltpu.VMEM((CRA * 128, DE), _f32),
            pltpu.VMEM((CRA * 128, DE), _f32),
            pltpu.VMEM((CRA * 128, DE), _f32),
            pltpu.VMEM_SHARED((N, DE), _f32),
            pltpu.VMEM_SHARED((N, DE), _f32),
            pltpu.VMEM_SHARED((N, DE), _f32),
            pltpu.SemaphoreType.DMA,
        ],
        compiler_params=pltpu.CompilerParams(use_tc_tiling_on_sc=False, needs_layout_passes=False),
    )
    return fn(src2, dst2, q2, psrc, pdst)


HD = D // 2
IRPS = IR // NS


def _lap_body(src2, dst2, w2, hl, hr, gout_l, gout_r, degout,
              sidx, didx, wv, hrows, zv, g_s, deg_s, sem):
    c = lax.axis_index("c")
    s = lax.axis_index("s")

    def _zb(i, carry):
        for cc in range(HD // 16):
            hrows[i, pl.ds(cc * 16, 16)] = jnp.zeros((16,), _f32)
        return carry
    lax.fori_loop(0, 1024, _zb, 0, unroll=4)

    def _zvb(i, carry):
        zv[pl.ds(i * 16, 16)] = jnp.zeros((16,), _f32)
        return carry
    lax.fori_loop(0, 64, _zvb, 0, unroll=8)

    pltpu.sync_copy(hrows.at[pl.ds(0, CP)], g_s.at[pl.ds(s * CP, CP)])

    @pl.when(s == 0)
    def _zrem():
        pltpu.sync_copy(hrows.at[pl.ds(0, REM)], g_s.at[pl.ds(NS * CP, REM)])

    @pl.when(c == 0)
    def _zdeg():
        pltpu.sync_copy(zv.at[pl.ds(0, DEG_PT)],
                        deg_s.at[pl.ds(s * DEG_PT, DEG_PT)])
    plsc.subcore_barrier()

    def _run(href, do_deg):
        for j in range(IRPS // 8):
            rbase = s * IRPS + j * 8
            pltpu.sync_copy(src2.at[pl.ds(rbase, 8)], sidx)
            pltpu.sync_copy(dst2.at[pl.ds(rbase, 8)], didx)
            pltpu.sync_copy(w2.at[pl.ds(rbase, 8)], wv)
            descs = []
            for r in range(8):
                descs.append(pltpu.async_copy(
                    href.at[sidx.at[r]],
                    hrows.at[pl.ds(r * 128, 128)], sem))
            for dsc in descs:
                dsc.wait()

            def _sb(e, carry):
                wspl = plsc.load_gather(
                    wv, [jnp.full((16,), e // 128, _i32),
                         jnp.full((16,), e % 128, _i32)])
                for cc in range(HD // 16):
                    hrows[e, pl.ds(cc * 16, 16)] = (
                        hrows[e, pl.ds(cc * 16, 16)] * wspl)
                return carry
            lax.fori_loop(0, 1024, _sb, 0)

            for r in range(8):
                pltpu.sync_copy(hrows.at[pl.ds(r * 128, 128)],
                                g_s.at[didx.at[r]], add=True)
                if do_deg:
                    pltpu.sync_copy(wv.at[r], deg_s.at[didx.at[r]], add=True)

    @pl.when(c == 0)
    def _run0():
        _run(hl, True)

    @pl.when(c == 1)
    def _run1():
        _run(hr, False)

    plsc.subcore_barrier()

    @pl.when(c == 0)
    def _c0out():
        pltpu.sync_copy(g_s.at[pl.ds(s * CP, CP)],
                        gout_l.at[pl.ds(s * CP, CP)])
        pltpu.sync_copy(deg_s.at[pl.ds(s * DEG_PT, DEG_PT)],
                        degout.at[pl.ds(s * DEG_PT, DEG_PT)])

        @pl.when(s == 0)
        def _crem0():
            pltpu.sync_copy(g_s.at[pl.ds(NS * CP, REM)],
                            gout_l.at[pl.ds(NS * CP, REM)])

    @pl.when(c == 1)
    def _c1out():
        pltpu.sync_copy(g_s.at[pl.ds(s * CP, CP)],
                        gout_r.at[pl.ds(s * CP, CP)])

        @pl.when(s == 0)
        def _crem1():
            pltpu.sync_copy(g_s.at[pl.ds(NS * CP, REM)],
                            gout_r.at[pl.ds(NS * CP, REM)])


def _lap_sc(src2, dst2, w2, hl, hr):
    mesh = plsc.VectorSubcoreMesh(core_axis_name="c", subcore_axis_name="s")
    fn = pl.kernel(
        _lap_body,
        out_type=[jax.ShapeDtypeStruct((N, HD), _f32),
                  jax.ShapeDtypeStruct((N, HD), _f32),
                  jax.ShapeDtypeStruct((DEG_PC,), _f32)],
        mesh=mesh,
        scratch_types=[
            pltpu.VMEM((8, 128), _i32),
            pltpu.VMEM((8, 128), _i32),
            pltpu.VMEM((8, 128), _f32),
            pltpu.VMEM((1024, HD), _f32),
            pltpu.VMEM((1024,), _f32),
            pltpu.VMEM_SHARED((N, HD), _f32),
            pltpu.VMEM_SHARED((DEG_PC,), _f32),
            pltpu.SemaphoreType.DMA,
        ],
        compiler_params=pltpu.CompilerParams(use_tc_tiling_on_sc=False, needs_layout_passes=False),
    )
    return fn(src2, dst2, w2, hl, hr)


BN1 = 2000
BE1 = 8192
BN2 = 2000


def _proj_body(xc_ref, wsd_ref, ps_ref, pd_ref):
    acc = jnp.dot(xc_ref[:], wsd_ref[:], preferred_element_type=_f32)
    ps_ref[:] = acc[:, :DE]
    pd_ref[:] = acc[:, DE:]


def _q_body(ec_ref, wq_ref, be_ref, q_ref):
    i = pl.program_id(0)
    acc = jnp.dot(ec_ref[:], wq_ref[:], preferred_element_type=_f32) + be_ref[:]
    rows = i * BE1 + lax.broadcasted_iota(_i32, (BE1, 1), 0)
    q_ref[:] = jnp.where(rows < E, acc, -1e30)


def _node_body(xc_ref, hb_ref, aa_ref, ab_ref, gl_ref, gr_ref,
               da_ref, wx_ref, wa_ref, bn_ref,
               xn_ref, td_ref, sp_ref):
    agg = aa_ref[:] + ab_ref[:]
    xnew = jnp.maximum(
        jnp.dot(xc_ref[:], wx_ref[:], preferred_element_type=_f32)
        + jnp.dot(agg, wa_ref[:], preferred_element_type=_f32)
        + bn_ref[:], 0.0)
    xn_ref[:] = xnew
    td_ref[:] = xnew - hb_ref[:]
    g = jnp.concatenate([gl_ref[:], gr_ref[:]], axis=1)
    sp_ref[:] = -COEFF * (da_ref[:] * hb_ref[:] - g)


def _proj_tc(xcat, wsd):
    return pl.pallas_call(
        _proj_body,
        grid=(N // BN1,),
        in_specs=[pl.BlockSpec((BN1, 2 * D), lambda i: (i, 0)),
                  pl.BlockSpec((2 * D, 2 * DE), lambda i: (0, 0))],
        out_specs=[pl.BlockSpec((BN1, DE), lambda i: (i, 0)),
                   pl.BlockSpec((BN1, DE), lambda i: (i, 0))],
        out_shape=[jax.ShapeDtypeStruct((N, DE), _f32),
                   jax.ShapeDtypeStruct((N, DE), _f32)],
    )(xcat, wsd)


def _q_tc(ecat_pad, wq, be):
    return pl.pallas_call(
        _q_body,
        grid=(E_PAD // BE1,),
        in_specs=[pl.BlockSpec((BE1, 2 * DE), lambda i: (i, 0)),
                  pl.BlockSpec((2 * DE, DE), lambda i: (0, 0)),
                  pl.BlockSpec((1, DE), lambda i: (0, 0))],
        out_specs=pl.BlockSpec((BE1, DE), lambda i: (i, 0)),
        out_shape=jax.ShapeDtypeStruct((E_PAD, DE), _f32),
    )(ecat_pad, wq, be)


def _node_tc(xcat, h, aggout, gl, gr, deg2, wx, wa, bn):
    nb = N // BN2
    return pl.pallas_call(
        _node_body,
        grid=(nb,),
        in_specs=[
            pl.BlockSpec((BN2, 2 * D), lambda i: (i, 0)),
            pl.BlockSpec((BN2, D), lambda i: (i, 0)),
            pl.BlockSpec((BN2, DE), lambda i: (i, 0)),
            pl.BlockSpec((BN2, DE), lambda i: (i + N // BN2, 0)),
            pl.BlockSpec((BN2, HD), lambda i: (i, 0)),
            pl.BlockSpec((BN2, HD), lambda i: (i, 0)),
            pl.BlockSpec((BN2, 1), lambda i: (i, 0)),
            pl.BlockSpec((2 * D, D), lambda i: (0, 0)),
            pl.BlockSpec((DE, D), lambda i: (0, 0)),
            pl.BlockSpec((1, D), lambda i: (0, 0)),
        ],
        out_specs=[pl.BlockSpec((BN2, D), lambda i: (i, 0)),
                   pl.BlockSpec((BN2, D), lambda i: (i, 0)),
                   pl.BlockSpec((BN2, D), lambda i: (i, 0))],
        out_shape=[jax.ShapeDtypeStruct((N, D), _f32),
                   jax.ShapeDtypeStruct((N, D), _f32),
                   jax.ShapeDtypeStruct((N, D), _f32)],
    )(xcat, h, aggout, aggout, gl, gr, deg2, wx, wa, bn)


def kernel(x, edge_index, edge_attr, h_init_x, h_init_edge_attr,
           lap_weight, W_e, b_e, W_n, b_n):
    xcat = jnp.concatenate([x, h_init_x], axis=1)
    wsd = jnp.concatenate([W_e[32:288], W_e[288:544]], axis=1)
    wq = W_e[:32]
    ecat = jnp.concatenate([edge_attr, h_init_edge_attr], axis=1)
    ecat_pad = jnp.pad(ecat, ((0, E_PAD - E), (0, 0)))

    src2 = jnp.pad(edge_index[0], (0, E_PAD - E)).reshape(IR, 128)
    dst2 = jnp.pad(edge_index[1], (0, E_PAD - E)).reshape(IR, 128)
    w2 = jnp.pad(lap_weight, (0, E_PAD - E)).reshape(IR, 128)

    psrc, pdst = _proj_tc(xcat, wsd)
    q2 = _q_tc(ecat_pad, wq, b_e.reshape(1, DE))
    enew_p, aggout = _edge_sc(src2, dst2, q2, psrc, pdst)
    gl, gr, degout = _lap_sc(src2, dst2, w2,
                             h_init_x[:, :HD], h_init_x[:, HD:])

    x_new, time_deriv, spatial_deriv = _node_tc(
        xcat, h_init_x, aggout, gl, gr, degout[:N].reshape(N, 1),
        W_n[:2 * D], W_n[2 * D:], b_n.reshape(1, D))

    return x_new, enew_p[:E], time_deriv, spatial_deriv


# speedup vs baseline: 4.2278x; 1.1176x over previous
import functools

import jax
import jax.numpy as jnp
from jax import lax
from jax.experimental import pallas as pl
from jax.experimental.pallas import tpu as pltpu
from jax.experimental.pallas import tpu_sc as plsc

N = 10000
E = 160000
D = 128
DE = 16
COEFF = 0.1

NC = 2
NS = 16
NW = NC * NS
EPW = 5120
E_PAD = NW * EPW
IR = E_PAD // 128
IRPW = IR // NW

CRA = 8
NCH_A = IRPW // CRA
CRB = 4
NCH_B = IRPW // CRB

CP = 624
REM = N - NS * CP
DEG_PC = 16000
DEG_PT = DEG_PC // NS

_f32 = jnp.float32
_i32 = jnp.int32


def _edge_body(src2, dst2, q2, psrc, pdst, enew, aggout,
               sidx, didx, qv, gs, gd, ev, agg_s, ps_s, pd_s, sem):
    c = lax.axis_index("c")
    s = lax.axis_index("s")
    wid = s * NC + c

    pltpu.sync_copy(psrc.at[pl.ds(s * CP, CP)], ps_s.at[pl.ds(s * CP, CP)])
    pltpu.sync_copy(pdst.at[pl.ds(s * CP, CP)], pd_s.at[pl.ds(s * CP, CP)])

    def _zb(i, carry):
        ev[i, :] = jnp.zeros((16,), _f32)
        return carry
    lax.fori_loop(0, CP, _zb, 0, unroll=8)
    pltpu.sync_copy(ev.at[pl.ds(0, CP)], agg_s.at[pl.ds(s * CP, CP)])

    @pl.when(s == 0)
    def _zrem():
        pltpu.sync_copy(ev.at[pl.ds(0, REM)], agg_s.at[pl.ds(NS * CP, REM)])
        pltpu.sync_copy(psrc.at[pl.ds(NS * CP, REM)],
                        ps_s.at[pl.ds(NS * CP, REM)])
        pltpu.sync_copy(pdst.at[pl.ds(NS * CP, REM)],
                        pd_s.at[pl.ds(NS * CP, REM)])
    plsc.subcore_barrier()

    for j in range(NCH_A):
        rbase = wid * IRPW + j * CRA
        ebase = rbase * 128
        pltpu.sync_copy(src2.at[pl.ds(rbase, CRA)], sidx)
        pltpu.sync_copy(dst2.at[pl.ds(rbase, CRA)], didx)
        pltpu.sync_copy(q2.at[pl.ds(ebase, CRA * 128)], qv)
        descs = []
        for r in range(CRA):
            descs.append(pltpu.async_copy(
                ps_s.at[sidx.at[r]], gs.at[pl.ds(r * 128, 128)], sem))
            descs.append(pltpu.async_copy(
                pd_s.at[didx.at[r]], gd.at[pl.ds(r * 128, 128)], sem))
        for dsc in descs:
            dsc.wait()

        def _cb(i, carry):
            ev[i, :] = jnp.maximum(qv[i, :] + gs[i, :] + gd[i, :], 0.0)
            return carry
        lax.fori_loop(0, CRA * 128, _cb, 0, unroll=4)

        pltpu.sync_copy(ev, enew.at[pl.ds(ebase, CRA * 128)])
        for r in range(CRA):
            pltpu.sync_copy(ev.at[pl.ds(r * 128, 128)],
                            agg_s.at[didx.at[r]], add=True)

    plsc.subcore_barrier()
    pltpu.sync_copy(agg_s.at[pl.ds(s * CP, CP)],
                    aggout.at[pl.ds(c * N + s * CP, CP)])

    @pl.when(s == 0)
    def _crem():
        pltpu.sync_copy(agg_s.at[pl.ds(NS * CP, REM)],
                        aggout.at[pl.ds(c * N + NS * CP, REM)])


def _edge_sc(src2, dst2, q2, psrc, pdst):
    mesh = plsc.VectorSubcoreMesh(core_axis_name="c", subcore_axis_name="s")
    fn = pl.kernel(
        _edge_body,
        out_type=[jax.ShapeDtypeStruct((E_PAD, DE), _f32),
                  jax.ShapeDtypeStruct((NC * N, DE), _f32)],
        mesh=mesh,
        scratch_types=[
            pltpu.VMEM((CRA, 128), _i32),
            pltpu.VMEM((CRA, 128), _i32),
            pltpu.VMEM((CRA * 128, DE), _f32),
            pltpu.VMEM((CRA * 128, DE), _f32),
            pltpu.VMEM((CRA * 128, DE), _f32),
            pltpu.VMEM((CRA * 128, DE), _f32),
            pltpu.VMEM_SHARED((N, DE), _f32),
            pltpu.VMEM_SHARED((N, DE), _f32),
            pltpu.VMEM_SHARED((N, DE), _f32),
            pltpu.SemaphoreType.DMA,
        ],
        compiler_params=pltpu.CompilerParams(use_tc_tiling_on_sc=False, needs_layout_passes=False),
    )
    return fn(src2, dst2, q2, psrc, pdst)


HD = D // 2
IRPS = IR // NS


def _lap_body(src2b, dst2, w2, h2, gout_l, gout_r, degout,
              sidx, didx, wv, hrows, zv, g_s, deg_s,
              gsem0, gsem1, ssem0, ssem1):
    c = lax.axis_index("c")
    s = lax.axis_index("s")

    def _zb(i, carry):
        for cc in range(HD // 16):
            hrows[i, pl.ds(cc * 16, 16)] = jnp.zeros((16,), _f32)
        return carry
    lax.fori_loop(0, 1024, _zb, 0, unroll=4)

    def _zvb(i, carry):
        zv[pl.ds(i * 16, 16)] = jnp.zeros((16,), _f32)
        return carry
    lax.fori_loop(0, 64, _zvb, 0, unroll=8)

    pltpu.sync_copy(hrows.at[pl.ds(0, CP)], g_s.at[pl.ds(s * CP, CP)])

    @pl.when(s == 0)
    def _zrem():
        pltpu.sync_copy(hrows.at[pl.ds(0, REM)], g_s.at[pl.ds(NS * CP, REM)])
    pltpu.sync_copy(zv.at[pl.ds(0, DEG_PT)],
                    deg_s.at[pl.ds(s * DEG_PT, DEG_PT)])
    plsc.subcore_barrier()

    gsems = (gsem0, gsem1)
    ssems = (ssem0, ssem1)
    NCH = IRPS // 4
    pend_g = [[], []]
    pend_s = [[], []]

    def load_chunk(j):
        b = j % 2
        r0 = 4 * b
        rbase = c * IR + s * IRPS + j * 4
        pltpu.sync_copy(src2b.at[pl.ds(rbase, 4)], sidx.at[pl.ds(r0, 4)])
        rbase2 = s * IRPS + j * 4
        pltpu.sync_copy(dst2.at[pl.ds(rbase2, 4)], didx.at[pl.ds(r0, 4)])
        pltpu.sync_copy(w2.at[pl.ds(rbase2, 4)], wv.at[pl.ds(r0, 4)])
        for r in range(4):
            pend_g[b].append(pltpu.async_copy(
                h2.at[sidx.at[r0 + r]],
                hrows.at[pl.ds((r0 + r) * 128, 128)], gsems[b]))

    load_chunk(0)
    for j in range(NCH):
        b = j % 2
        r0 = 4 * b
        if j + 1 < NCH:
            for dsc in pend_s[1 - b]:
                dsc.wait()
            pend_s[1 - b] = []
            load_chunk(j + 1)
        for dsc in pend_g[b]:
            dsc.wait()
        pend_g[b] = []

        def _sb(e, carry):
            wspl = plsc.load_gather(
                wv, [jnp.full((16,), r0 + e // 128, _i32),
                     jnp.full((16,), e % 128, _i32)])
            for cc in range(HD // 16):
                hrows[r0 * 128 + e, pl.ds(cc * 16, 16)] = (
                    hrows[r0 * 128 + e, pl.ds(cc * 16, 16)] * wspl)
            return carry
        lax.fori_loop(0, 512, _sb, 0)

        for r in range(4):
            pend_s[b].append(pltpu.async_copy(
                hrows.at[pl.ds((r0 + r) * 128, 128)],
                g_s.at[didx.at[r0 + r]], ssems[b], add=True))
            pend_s[b].append(pltpu.async_copy(
                wv.at[r0 + r], deg_s.at[didx.at[r0 + r]], ssems[b],
                add=True))
    for b in range(2):
        for dsc in pend_s[b]:
            dsc.wait()

    plsc.subcore_barrier()

    @pl.when(c == 0)
    def _c0out():
        pltpu.sync_copy(g_s.at[pl.ds(s * CP, CP)],
                        gout_l.at[pl.ds(s * CP, CP)])
        pltpu.sync_copy(deg_s.at[pl.ds(s * DEG_PT, DEG_PT)],
                        degout.at[pl.ds(s * DEG_PT, DEG_PT)])

        @pl.when(s == 0)
        def _crem0():
            pltpu.sync_copy(g_s.at[pl.ds(NS * CP, REM)],
                            gout_l.at[pl.ds(NS * CP, REM)])

    @pl.when(c == 1)
    def _c1out():
        pltpu.sync_copy(g_s.at[pl.ds(s * CP, CP)],
                        gout_r.at[pl.ds(s * CP, CP)])

        @pl.when(s == 0)
        def _crem1():
            pltpu.sync_copy(g_s.at[pl.ds(NS * CP, REM)],
                            gout_r.at[pl.ds(NS * CP, REM)])


def _lap_sc(src2b, dst2, w2, h2):
    mesh = plsc.VectorSubcoreMesh(core_axis_name="c", subcore_axis_name="s")
    fn = pl.kernel(
        _lap_body,
        out_type=[jax.ShapeDtypeStruct((N, HD), _f32),
                  jax.ShapeDtypeStruct((N, HD), _f32),
                  jax.ShapeDtypeStruct((DEG_PC,), _f32)],
        mesh=mesh,
        scratch_types=[
            pltpu.VMEM((8, 128), _i32),
            pltpu.VMEM((8, 128), _i32),
            pltpu.VMEM((8, 128), _f32),
            pltpu.VMEM((1024, HD), _f32),
            pltpu.VMEM((1024,), _f32),
            pltpu.VMEM_SHARED((N, HD), _f32),
            pltpu.VMEM_SHARED((DEG_PC,), _f32),
            pltpu.SemaphoreType.DMA,
            pltpu.SemaphoreType.DMA,
            pltpu.SemaphoreType.DMA,
            pltpu.SemaphoreType.DMA,
        ],
        compiler_params=pltpu.CompilerParams(use_tc_tiling_on_sc=False, needs_layout_passes=False),
    )
    return fn(src2b, dst2, w2, h2)


BN1 = 2000
BE1 = 8192
BN2 = 2000


def _proj_body(xc_ref, wsd_ref, ps_ref, pd_ref):
    acc = jnp.dot(xc_ref[:], wsd_ref[:], preferred_element_type=_f32)
    ps_ref[:] = acc[:, :DE]
    pd_ref[:] = acc[:, DE:]


def _q_body(ec_ref, wq_ref, be_ref, q_ref):
    i = pl.program_id(0)
    acc = jnp.dot(ec_ref[:], wq_ref[:], preferred_element_type=_f32) + be_ref[:]
    rows = i * BE1 + lax.broadcasted_iota(_i32, (BE1, 1), 0)
    q_ref[:] = jnp.where(rows < E, acc, -1e30)


def _node_body(xc_ref, hb_ref, aa_ref, ab_ref, gl_ref, gr_ref,
               da_ref, wx_ref, wa_ref, bn_ref,
               xn_ref, td_ref, sp_ref):
    agg = aa_ref[:] + ab_ref[:]
    xnew = jnp.maximum(
        jnp.dot(xc_ref[:], wx_ref[:], preferred_element_type=_f32)
        + jnp.dot(agg, wa_ref[:], preferred_element_type=_f32)
        + bn_ref[:], 0.0)
    xn_ref[:] = xnew
    td_ref[:] = xnew - hb_ref[:]
    g = jnp.concatenate([gl_ref[:], gr_ref[:]], axis=1)
    sp_ref[:] = -COEFF * (da_ref[:] * hb_ref[:] - g)


def _proj_tc(xcat, wsd):
    return pl.pallas_call(
        _proj_body,
        grid=(N // BN1,),
        in_specs=[pl.BlockSpec((BN1, 2 * D), lambda i: (i, 0)),
                  pl.BlockSpec((2 * D, 2 * DE), lambda i: (0, 0))],
        out_specs=[pl.BlockSpec((BN1, DE), lambda i: (i, 0)),
                   pl.BlockSpec((BN1, DE), lambda i: (i, 0))],
        out_shape=[jax.ShapeDtypeStruct((N, DE), _f32),
                   jax.ShapeDtypeStruct((N, DE), _f32)],
    )(xcat, wsd)


def _q_tc(ecat_pad, wq, be):
    return pl.pallas_call(
        _q_body,
        grid=(E_PAD // BE1,),
        in_specs=[pl.BlockSpec((BE1, 2 * DE), lambda i: (i, 0)),
                  pl.BlockSpec((2 * DE, DE), lambda i: (0, 0)),
                  pl.BlockSpec((1, DE), lambda i: (0, 0))],
        out_specs=pl.BlockSpec((BE1, DE), lambda i: (i, 0)),
        out_shape=jax.ShapeDtypeStruct((E_PAD, DE), _f32),
    )(ecat_pad, wq, be)


def _node_tc(xcat, h, aggout, gl, gr, deg2, wx, wa, bn):
    nb = N // BN2
    return pl.pallas_call(
        _node_body,
        grid=(nb,),
        in_specs=[
            pl.BlockSpec((BN2, 2 * D), lambda i: (i, 0)),
            pl.BlockSpec((BN2, D), lambda i: (i, 0)),
            pl.BlockSpec((BN2, DE), lambda i: (i, 0)),
            pl.BlockSpec((BN2, DE), lambda i: (i + N // BN2, 0)),
            pl.BlockSpec((BN2, HD), lambda i: (i, 0)),
            pl.BlockSpec((BN2, HD), lambda i: (i, 0)),
            pl.BlockSpec((BN2, 1), lambda i: (i, 0)),
            pl.BlockSpec((2 * D, D), lambda i: (0, 0)),
            pl.BlockSpec((DE, D), lambda i: (0, 0)),
            pl.BlockSpec((1, D), lambda i: (0, 0)),
        ],
        out_specs=[pl.BlockSpec((BN2, D), lambda i: (i, 0)),
                   pl.BlockSpec((BN2, D), lambda i: (i, 0)),
                   pl.BlockSpec((BN2, D), lambda i: (i, 0))],
        out_shape=[jax.ShapeDtypeStruct((N, D), _f32),
                   jax.ShapeDtypeStruct((N, D), _f32),
                   jax.ShapeDtypeStruct((N, D), _f32)],
    )(xcat, h, aggout, aggout, gl, gr, deg2, wx, wa, bn)


def kernel(x, edge_index, edge_attr, h_init_x, h_init_edge_attr,
           lap_weight, W_e, b_e, W_n, b_n):
    xcat = jnp.concatenate([x, h_init_x], axis=1)
    wsd = jnp.concatenate([W_e[32:288], W_e[288:544]], axis=1)
    wq = W_e[:32]
    ecat = jnp.concatenate([edge_attr, h_init_edge_attr], axis=1)
    ecat_pad = jnp.pad(ecat, ((0, E_PAD - E), (0, 0)))

    src2 = jnp.pad(edge_index[0], (0, E_PAD - E)).reshape(IR, 128)
    dst2 = jnp.pad(edge_index[1], (0, E_PAD - E)).reshape(IR, 128)
    w2 = jnp.pad(lap_weight, (0, E_PAD - E)).reshape(IR, 128)

    psrc, pdst = _proj_tc(xcat, wsd)
    q2 = _q_tc(ecat_pad, wq, b_e.reshape(1, DE))
    enew_p, aggout = _edge_sc(src2, dst2, q2, psrc, pdst)
    src2b = jnp.concatenate([src2, src2 + N], axis=0)
    h2 = jnp.concatenate([h_init_x[:, :HD], h_init_x[:, HD:]], axis=0)
    gl, gr, degout = _lap_sc(src2b, dst2, w2, h2)

    x_new, time_deriv, spatial_deriv = _node_tc(
        xcat, h_init_x, aggout, gl, gr, degout[:N].reshape(N, 1),
        W_n[:2 * D], W_n[2 * D:], b_n.reshape(1, D))

    return x_new, enew_p[:E], time_deriv, spatial_deriv


# speedup vs baseline: 4.6606x; 1.1024x over previous
import jax
import jax.numpy as jnp
from jax import lax
from jax.experimental import pallas as pl
from jax.experimental.pallas import tpu as pltpu
from jax.experimental.pallas import tpu_sc as plsc

N = 10000
E = 160000
D = 128
DE = 16
HD = D // 2
COEFF = 0.1

NC = 2
NS = 16
NW = NC * NS
IR = E // 128

CP = 624
REM = N - NS * CP
DEG_PC = 16000
DEG_PT = DEG_PC // NS

_f32 = jnp.float32
_i32 = jnp.int32

_SC_PARAMS = pltpu.CompilerParams(use_tc_tiling_on_sc=False,
                                  needs_layout_passes=False)


def _edge_body(src2, dst2, q2, psrc, pdst, enew, aggout,
               sidx, didx, qv, gs, gd, ev, agg_s, ps_s, pd_s, sem):
    c = lax.axis_index("c")
    s = lax.axis_index("s")
    wid = s * NC + c

    pltpu.sync_copy(psrc.at[pl.ds(s * CP, CP)], ps_s.at[pl.ds(s * CP, CP)])
    pltpu.sync_copy(pdst.at[pl.ds(s * CP, CP)], pd_s.at[pl.ds(s * CP, CP)])

    def _zb(i, carry):
        ev[i, :] = jnp.zeros((16,), _f32)
        return carry
    lax.fori_loop(0, CP, _zb, 0, unroll=8)
    pltpu.sync_copy(ev.at[pl.ds(0, CP)], agg_s.at[pl.ds(s * CP, CP)])

    @pl.when(s == 0)
    def _zrem():
        pltpu.sync_copy(ev.at[pl.ds(0, REM)], agg_s.at[pl.ds(NS * CP, REM)])
        pltpu.sync_copy(psrc.at[pl.ds(NS * CP, REM)],
                        ps_s.at[pl.ds(NS * CP, REM)])
        pltpu.sync_copy(pdst.at[pl.ds(NS * CP, REM)],
                        pd_s.at[pl.ds(NS * CP, REM)])
    plsc.subcore_barrier()

    def do_chunk(rbase, nr):
        ebase = rbase * 128
        ne = nr * 128
        pltpu.sync_copy(src2.at[pl.ds(rbase, nr)], sidx.at[pl.ds(0, nr)])
        pltpu.sync_copy(dst2.at[pl.ds(rbase, nr)], didx.at[pl.ds(0, nr)])
        pltpu.sync_copy(q2.at[pl.ds(ebase, ne)], qv.at[pl.ds(0, ne)])
        descs = []
        for r in range(nr):
            descs.append(pltpu.async_copy(
                ps_s.at[sidx.at[r]], gs.at[pl.ds(r * 128, 128)], sem))
            descs.append(pltpu.async_copy(
                pd_s.at[didx.at[r]], gd.at[pl.ds(r * 128, 128)], sem))
        for dsc in descs:
            dsc.wait()

        def _cb(i, carry):
            ev[i, :] = jnp.maximum(qv[i, :] + gs[i, :] + gd[i, :], 0.0)
            return carry
        lax.fori_loop(0, ne, _cb, 0, unroll=4)

        pltpu.sync_copy(ev.at[pl.ds(0, ne)], enew.at[pl.ds(ebase, ne)])
        for r in range(nr):
            pltpu.sync_copy(ev.at[pl.ds(r * 128, 128)],
                            agg_s.at[didx.at[r]], add=True)

    @pl.when(wid != NW - 1)
    def _main():
        for j in range(5):
            do_chunk(wid * 40 + j * 8, 8)

    @pl.when(wid == NW - 1)
    def _tail():
        do_chunk(1240, 8)
        do_chunk(1248, 2)

    plsc.subcore_barrier()
    pltpu.sync_copy(agg_s.at[pl.ds(s * CP, CP)],
                    aggout.at[pl.ds(c * N + s * CP, CP)])

    @pl.when(s == 0)
    def _crem():
        pltpu.sync_copy(agg_s.at[pl.ds(NS * CP, REM)],
                        aggout.at[pl.ds(c * N + NS * CP, REM)])


def _edge_sc(src2, dst2, q2, psrc, pdst):
    mesh = plsc.VectorSubcoreMesh(core_axis_name="c", subcore_axis_name="s")
    fn = pl.kernel(
        _edge_body,
        out_type=[jax.ShapeDtypeStruct((E, DE), _f32),
                  jax.ShapeDtypeStruct((NC * N, DE), _f32)],
        mesh=mesh,
        scratch_types=[
            pltpu.VMEM((8, 128), _i32),
            pltpu.VMEM((8, 128), _i32),
            pltpu.VMEM((1024, DE), _f32),
            pltpu.VMEM((1024, DE), _f32),
            pltpu.VMEM((1024, DE), _f32),
            pltpu.VMEM((1024, DE), _f32),
            pltpu.VMEM_SHARED((N, DE), _f32),
            pltpu.VMEM_SHARED((N, DE), _f32),
            pltpu.VMEM_SHARED((N, DE), _f32),
            pltpu.SemaphoreType.DMA,
        ],
        compiler_params=_SC_PARAMS,
    )
    return fn(src2, dst2, q2, psrc, pdst)


def _lap_body(src2b, dst2, w2, h2, gout_l, gout_r, degout,
              sidx, didx, wv, hrows, zv, g_s, deg_s,
              gsem0, gsem1, ssem0, ssem1):
    c = lax.axis_index("c")
    s = lax.axis_index("s")

    def _zb(i, carry):
        for cc in range(HD // 16):
            hrows[i, pl.ds(cc * 16, 16)] = jnp.zeros((16,), _f32)
        return carry
    lax.fori_loop(0, 1024, _zb, 0, unroll=4)

    def _zvb(i, carry):
        zv[pl.ds(i * 16, 16)] = jnp.zeros((16,), _f32)
        return carry
    lax.fori_loop(0, 64, _zvb, 0, unroll=8)

    pltpu.sync_copy(hrows.at[pl.ds(0, CP)], g_s.at[pl.ds(s * CP, CP)])

    @pl.when(s == 0)
    def _zrem():
        pltpu.sync_copy(hrows.at[pl.ds(0, REM)], g_s.at[pl.ds(NS * CP, REM)])
    pltpu.sync_copy(zv.at[pl.ds(0, DEG_PT)],
                    deg_s.at[pl.ds(s * DEG_PT, DEG_PT)])
    plsc.subcore_barrier()

    gsems = (gsem0, gsem1)
    ssems = (ssem0, ssem1)

    def run(chunks):
        pend_g = [[], []]
        pend_s = [[], []]

        def load_chunk(i):
            rb, nr = chunks[i]
            b = i % 2
            r0 = 4 * b
            pltpu.sync_copy(src2b.at[pl.ds(c * IR + rb, nr)],
                            sidx.at[pl.ds(r0, nr)])
            pltpu.sync_copy(dst2.at[pl.ds(rb, nr)], didx.at[pl.ds(r0, nr)])
            pltpu.sync_copy(w2.at[pl.ds(rb, nr)], wv.at[pl.ds(r0, nr)])
            for r in range(nr):
                pend_g[b].append(pltpu.async_copy(
                    h2.at[sidx.at[r0 + r]],
                    hrows.at[pl.ds((r0 + r) * 128, 128)], gsems[b]))

        load_chunk(0)
        for i in range(len(chunks)):
            rb, nr = chunks[i]
            b = i % 2
            r0 = 4 * b
            if i + 1 < len(chunks):
                for dsc in pend_s[1 - b]:
                    dsc.wait()
                pend_s[1 - b] = []
                load_chunk(i + 1)
            for dsc in pend_g[b]:
                dsc.wait()
            pend_g[b] = []

            def _sb(e, carry):
                wspl = plsc.load_gather(
                    wv, [jnp.full((16,), r0 + e // 128, _i32),
                         jnp.full((16,), e % 128, _i32)])
                for cc in range(HD // 16):
                    hrows[r0 * 128 + e, pl.ds(cc * 16, 16)] = (
                        hrows[r0 * 128 + e, pl.ds(cc * 16, 16)] * wspl)
                return carry
            lax.fori_loop(0, nr * 128, _sb, 0)

            for r in range(nr):
                pend_s[b].append(pltpu.async_copy(
                    hrows.at[pl.ds((r0 + r) * 128, 128)],
                    g_s.at[didx.at[r0 + r]], ssems[b], add=True))
                pend_s[b].append(pltpu.async_copy(
                    wv.at[r0 + r], deg_s.at[didx.at[r0 + r]], ssems[b],
                    add=True))
        for b in range(2):
            for dsc in pend_s[b]:
                dsc.wait()

    @pl.when(s != NS - 1)
    def _main():
        run([(s * 80 + i * 4, 4) for i in range(20)])

    @pl.when(s == NS - 1)
    def _tail():
        run([(1200 + i * 4, 4) for i in range(12)] + [(1248, 2)])

    plsc.subcore_barrier()

    @pl.when(c == 0)
    def _c0out():
        pltpu.sync_copy(g_s.at[pl.ds(s * CP, CP)],
                        gout_l.at[pl.ds(s * CP, CP)])
        pltpu.sync_copy(deg_s.at[pl.ds(s * DEG_PT, DEG_PT)],
                        degout.at[pl.ds(s * DEG_PT, DEG_PT)])

        @pl.when(s == 0)
        def _crem0():
            pltpu.sync_copy(g_s.at[pl.ds(NS * CP, REM)],
                            gout_l.at[pl.ds(NS * CP, REM)])

    @pl.when(c == 1)
    def _c1out():
        pltpu.sync_copy(g_s.at[pl.ds(s * CP, CP)],
                        gout_r.at[pl.ds(s * CP, CP)])

        @pl.when(s == 0)
        def _crem1():
            pltpu.sync_copy(g_s.at[pl.ds(NS * CP, REM)],
                            gout_r.at[pl.ds(NS * CP, REM)])


def _lap_sc(src2b, dst2, w2, h2):
    mesh = plsc.VectorSubcoreMesh(core_axis_name="c", subcore_axis_name="s")
    fn = pl.kernel(
        _lap_body,
        out_type=[jax.ShapeDtypeStruct((N, HD), _f32),
                  jax.ShapeDtypeStruct((N, HD), _f32),
                  jax.ShapeDtypeStruct((DEG_PC,), _f32)],
        mesh=mesh,
        scratch_types=[
            pltpu.VMEM((8, 128), _i32),
            pltpu.VMEM((8, 128), _i32),
            pltpu.VMEM((8, 128), _f32),
            pltpu.VMEM((1024, HD), _f32),
            pltpu.VMEM((1024,), _f32),
            pltpu.VMEM_SHARED((N, HD), _f32),
            pltpu.VMEM_SHARED((DEG_PC,), _f32),
            pltpu.SemaphoreType.DMA,
            pltpu.SemaphoreType.DMA,
            pltpu.SemaphoreType.DMA,
            pltpu.SemaphoreType.DMA,
        ],
        compiler_params=_SC_PARAMS,
    )
    return fn(src2b, dst2, w2, h2)


BN1 = 2000
BE1 = 8000
BN2 = 2000


def _proj_body(x_ref, h_ref, wx_ref, wh_ref, ps_ref, pd_ref):
    acc = (jnp.dot(x_ref[:], wx_ref[:], preferred_element_type=_f32)
           + jnp.dot(h_ref[:], wh_ref[:], preferred_element_type=_f32))
    ps_ref[:] = acc[:, :DE]
    pd_ref[:] = acc[:, DE:]


def _q_body(ea_ref, he_ref, w1_ref, w2_ref, be_ref, q_ref):
    q_ref[:] = (jnp.dot(ea_ref[:], w1_ref[:], preferred_element_type=_f32)
                + jnp.dot(he_ref[:], w2_ref[:], preferred_element_type=_f32)
                + be_ref[:])


def _node_body(x_ref, hb_ref, aa_ref, ab_ref, gl_ref, gr_ref,
               da_ref, wx_ref, wh_ref, wa_ref, bn_ref,
               xn_ref, td_ref, sp_ref):
    agg = aa_ref[:] + ab_ref[:]
    xnew = jnp.maximum(
        jnp.dot(x_ref[:], wx_ref[:], preferred_element_type=_f32)
        + jnp.dot(hb_ref[:], wh_ref[:], preferred_element_type=_f32)
        + jnp.dot(agg, wa_ref[:], preferred_element_type=_f32)
        + bn_ref[:], 0.0)
    xn_ref[:] = xnew
    td_ref[:] = xnew - hb_ref[:]
    g = jnp.concatenate([gl_ref[:], gr_ref[:]], axis=1)
    sp_ref[:] = -COEFF * (da_ref[:] * hb_ref[:] - g)


def _proj_tc(x, h, wx, wh):
    return pl.pallas_call(
        _proj_body,
        grid=(N // BN1,),
        in_specs=[pl.BlockSpec((BN1, D), lambda i: (i, 0)),
                  pl.BlockSpec((BN1, D), lambda i: (i, 0)),
                  pl.BlockSpec((D, 2 * DE), lambda i: (0, 0)),
                  pl.BlockSpec((D, 2 * DE), lambda i: (0, 0))],
        out_specs=[pl.BlockSpec((BN1, DE), lambda i: (i, 0)),
                   pl.BlockSpec((BN1, DE), lambda i: (i, 0))],
        out_shape=[jax.ShapeDtypeStruct((N, DE), _f32),
                   jax.ShapeDtypeStruct((N, DE), _f32)],
    )(x, h, wx, wh)


def _q_tc(ea, he, w1, w2, be):
    return pl.pallas_call(
        _q_body,
        grid=(E // BE1,),
        in_specs=[pl.BlockSpec((BE1, DE), lambda i: (i, 0)),
                  pl.BlockSpec((BE1, DE), lambda i: (i, 0)),
                  pl.BlockSpec((DE, DE), lambda i: (0, 0)),
                  pl.BlockSpec((DE, DE), lambda i: (0, 0)),
                  pl.BlockSpec((1, DE), lambda i: (0, 0))],
        out_specs=pl.BlockSpec((BE1, DE), lambda i: (i, 0)),
        out_shape=jax.ShapeDtypeStruct((E, DE), _f32),
    )(ea, he, w1, w2, be)


def _node_tc(x, h, aggout, gl, gr, deg2, wx, wh, wa, bn):
    return pl.pallas_call(
        _node_body,
        grid=(N // BN2,),
        in_specs=[
            pl.BlockSpec((BN2, D), lambda i: (i, 0)),
            pl.BlockSpec((BN2, D), lambda i: (i, 0)),
            pl.BlockSpec((BN2, DE), lambda i: (i, 0)),
            pl.BlockSpec((BN2, DE), lambda i: (i + N // BN2, 0)),
            pl.BlockSpec((BN2, HD), lambda i: (i, 0)),
            pl.BlockSpec((BN2, HD), lambda i: (i, 0)),
            pl.BlockSpec((BN2, 1), lambda i: (i, 0)),
            pl.BlockSpec((D, D), lambda i: (0, 0)),
            pl.BlockSpec((D, D), lambda i: (0, 0)),
            pl.BlockSpec((DE, D), lambda i: (0, 0)),
            pl.BlockSpec((1, D), lambda i: (0, 0)),
        ],
        out_specs=[pl.BlockSpec((BN2, D), lambda i: (i, 0)),
                   pl.BlockSpec((BN2, D), lambda i: (i, 0)),
                   pl.BlockSpec((BN2, D), lambda i: (i, 0))],
        out_shape=[jax.ShapeDtypeStruct((N, D), _f32),
                   jax.ShapeDtypeStruct((N, D), _f32),
                   jax.ShapeDtypeStruct((N, D), _f32)],
    )(x, h, aggout, aggout, gl, gr, deg2, wx, wh, wa, bn)


def kernel(x, edge_index, edge_attr, h_init_x, h_init_edge_attr,
           lap_weight, W_e, b_e, W_n, b_n):
    src2 = edge_index[0].reshape(IR, 128)
    dst2 = edge_index[1].reshape(IR, 128)
    w2 = lap_weight.reshape(IR, 128)
    src2b = jnp.concatenate([src2 * 2, src2 * 2 + 1], axis=0)
    h2 = h_init_x.reshape(2 * N, HD)

    gl, gr, degout = _lap_sc(src2b, dst2, w2, h2)

    wp_x = jnp.concatenate([W_e[32:160], W_e[288:416]], axis=1)
    wp_h = jnp.concatenate([W_e[160:288], W_e[416:544]], axis=1)
    psrc, pdst = _proj_tc(x, h_init_x, wp_x, wp_h)
    q2 = _q_tc(edge_attr, h_init_edge_attr, W_e[:DE], W_e[DE:2 * DE],
               b_e.reshape(1, DE))
    enew, aggout = _edge_sc(src2, dst2, q2, psrc, pdst)

    x_new, time_deriv, spatial_deriv = _node_tc(
        x, h_init_x, aggout, gl, gr, degout[:N].reshape(N, 1),
        W_n[:D], W_n[D:2 * D], W_n[2 * D:], b_n.reshape(1, D))

    return x_new, enew, time_deriv, spatial_deriv


# speedup vs baseline: 5.5806x; 1.1974x over previous
import jax
import jax.numpy as jnp
from jax import lax
from jax.experimental import pallas as pl
from jax.experimental.pallas import tpu as pltpu
from jax.experimental.pallas import tpu_sc as plsc

N = 10000
E = 160000
D = 128
DE = 16
HD = D // 2
COEFF = 0.1

NC = 2
NS = 16
NW = NC * NS
IR = E // 128

CP = 624
REM = N - NS * CP
DEG_PC = 16000
DEG_PT = DEG_PC // NS

_f32 = jnp.float32
_i32 = jnp.int32

_SC_PARAMS = pltpu.CompilerParams(use_tc_tiling_on_sc=False,
                                  needs_layout_passes=False)


def _edge_body(src2, dst2, q2, psrc, pdst, enew, aggout,
               sidx, didx, qv, gs, gd, ev, agg_s, ps_s, pd_s, sem):
    c = lax.axis_index("c")
    s = lax.axis_index("s")
    wid = s * NC + c

    pltpu.sync_copy(psrc.at[pl.ds(s * CP, CP)], ps_s.at[pl.ds(s * CP, CP)])
    pltpu.sync_copy(pdst.at[pl.ds(s * CP, CP)], pd_s.at[pl.ds(s * CP, CP)])

    def _zb(i, carry):
        ev[i, :] = jnp.zeros((16,), _f32)
        return carry
    lax.fori_loop(0, CP, _zb, 0, unroll=8)
    pltpu.sync_copy(ev.at[pl.ds(0, CP)], agg_s.at[pl.ds(s * CP, CP)])

    @pl.when(s == 0)
    def _zrem():
        pltpu.sync_copy(ev.at[pl.ds(0, REM)], agg_s.at[pl.ds(NS * CP, REM)])
        pltpu.sync_copy(psrc.at[pl.ds(NS * CP, REM)],
                        ps_s.at[pl.ds(NS * CP, REM)])
        pltpu.sync_copy(pdst.at[pl.ds(NS * CP, REM)],
                        pd_s.at[pl.ds(NS * CP, REM)])
    plsc.subcore_barrier()

    def do_chunk(rbase, nr):
        ebase = rbase * 128
        ne = nr * 128
        pltpu.sync_copy(src2.at[pl.ds(rbase, nr)], sidx.at[pl.ds(0, nr)])
        pltpu.sync_copy(dst2.at[pl.ds(rbase, nr)], didx.at[pl.ds(0, nr)])
        pltpu.sync_copy(q2.at[pl.ds(rbase * 16, nr * 16)],
                        qv.at[pl.ds(0, nr * 16)])
        descs = []
        for r in range(nr):
            descs.append(pltpu.async_copy(
                ps_s.at[sidx.at[r]], gs.at[pl.ds(r * 128, 128)], sem))
            descs.append(pltpu.async_copy(
                pd_s.at[didx.at[r]], gd.at[pl.ds(r * 128, 128)], sem))
        for dsc in descs:
            dsc.wait()

        def _cb(i, carry):
            ev[i, :] = jnp.maximum(
                qv[i // 8, pl.ds((i % 8) * 16, 16)] + gs[i, :] + gd[i, :],
                0.0)
            return carry
        lax.fori_loop(0, ne, _cb, 0, unroll=4)

        pltpu.sync_copy(ev.at[pl.ds(0, ne)], enew.at[pl.ds(ebase, ne)])
        for r in range(nr):
            pltpu.sync_copy(ev.at[pl.ds(r * 128, 128)],
                            agg_s.at[didx.at[r]], add=True)

    @pl.when(wid != NW - 1)
    def _main():
        for j in range(5):
            do_chunk(wid * 40 + j * 8, 8)

    @pl.when(wid == NW - 1)
    def _tail():
        do_chunk(1240, 8)
        do_chunk(1248, 2)

    plsc.subcore_barrier()
    pltpu.sync_copy(agg_s.at[pl.ds(s * CP, CP)],
                    aggout.at[pl.ds(c * N + s * CP, CP)])

    @pl.when(s == 0)
    def _crem():
        pltpu.sync_copy(agg_s.at[pl.ds(NS * CP, REM)],
                        aggout.at[pl.ds(c * N + NS * CP, REM)])


def _edge_sc(src2, dst2, q2, psrc, pdst):
    mesh = plsc.VectorSubcoreMesh(core_axis_name="c", subcore_axis_name="s")
    fn = pl.kernel(
        _edge_body,
        out_type=[jax.ShapeDtypeStruct((E, DE), _f32),
                  jax.ShapeDtypeStruct((NC * N, DE), _f32)],
        mesh=mesh,
        scratch_types=[
            pltpu.VMEM((8, 128), _i32),
            pltpu.VMEM((8, 128), _i32),
            pltpu.VMEM((128, 128), _f32),
            pltpu.VMEM((1024, DE), _f32),
            pltpu.VMEM((1024, DE), _f32),
            pltpu.VMEM((1024, DE), _f32),
            pltpu.VMEM_SHARED((N, DE), _f32),
            pltpu.VMEM_SHARED((N, DE), _f32),
            pltpu.VMEM_SHARED((N, DE), _f32),
            pltpu.SemaphoreType.DMA,
        ],
        compiler_params=_SC_PARAMS,
    )
    return fn(src2, dst2, q2, psrc, pdst)


def _lap_body(src2b, dst2, w2, h2, gout_l, gout_r, degout,
              sidx, didx, wv, hrows, zv, g_s, deg_s,
              gsem0, gsem1, ssem0, ssem1):
    c = lax.axis_index("c")
    s = lax.axis_index("s")

    def _zb(i, carry):
        for cc in range(HD // 16):
            hrows[i, pl.ds(cc * 16, 16)] = jnp.zeros((16,), _f32)
        return carry
    lax.fori_loop(0, 1024, _zb, 0, unroll=4)

    def _zvb(i, carry):
        zv[pl.ds(i * 16, 16)] = jnp.zeros((16,), _f32)
        return carry
    lax.fori_loop(0, 64, _zvb, 0, unroll=8)

    pltpu.sync_copy(hrows.at[pl.ds(0, CP)], g_s.at[pl.ds(s * CP, CP)])

    @pl.when(s == 0)
    def _zrem():
        pltpu.sync_copy(hrows.at[pl.ds(0, REM)], g_s.at[pl.ds(NS * CP, REM)])
    pltpu.sync_copy(zv.at[pl.ds(0, DEG_PT)],
                    deg_s.at[pl.ds(s * DEG_PT, DEG_PT)])
    plsc.subcore_barrier()

    gsems = (gsem0, gsem1)
    ssems = (ssem0, ssem1)

    def run(chunks):
        pend_g = [[], []]
        pend_s = [[], []]

        def load_chunk(i):
            rb, nr = chunks[i]
            b = i % 2
            r0 = 4 * b
            pltpu.sync_copy(src2b.at[pl.ds(c * IR + rb, nr)],
                            sidx.at[pl.ds(r0, nr)])
            pltpu.sync_copy(dst2.at[pl.ds(rb, nr)], didx.at[pl.ds(r0, nr)])
            pltpu.sync_copy(w2.at[pl.ds(rb, nr)], wv.at[pl.ds(r0, nr)])
            for r in range(nr):
                pend_g[b].append(pltpu.async_copy(
                    h2.at[sidx.at[r0 + r]],
                    hrows.at[pl.ds((r0 + r) * 128, 128)], gsems[b]))

        load_chunk(0)
        for i in range(len(chunks)):
            rb, nr = chunks[i]
            b = i % 2
            r0 = 4 * b
            if i + 1 < len(chunks):
                for dsc in pend_s[1 - b]:
                    dsc.wait()
                pend_s[1 - b] = []
                load_chunk(i + 1)
            for dsc in pend_g[b]:
                dsc.wait()
            pend_g[b] = []

            def _sb(e, carry):
                wspl = plsc.load_gather(
                    wv, [jnp.full((16,), r0 + e // 128, _i32),
                         jnp.full((16,), e % 128, _i32)])
                for cc in range(HD // 16):
                    hrows[r0 * 128 + e, pl.ds(cc * 16, 16)] = (
                        hrows[r0 * 128 + e, pl.ds(cc * 16, 16)] * wspl)
                return carry
            lax.fori_loop(0, nr * 128, _sb, 0)

            for r in range(nr):
                pend_s[b].append(pltpu.async_copy(
                    hrows.at[pl.ds((r0 + r) * 128, 128)],
                    g_s.at[didx.at[r0 + r]], ssems[b], add=True))
                pend_s[b].append(pltpu.async_copy(
                    wv.at[r0 + r], deg_s.at[didx.at[r0 + r]], ssems[b],
                    add=True))
        for b in range(2):
            for dsc in pend_s[b]:
                dsc.wait()

    @pl.when(s != NS - 1)
    def _main():
        run([(s * 80 + i * 4, 4) for i in range(20)])

    @pl.when(s == NS - 1)
    def _tail():
        run([(1200 + i * 4, 4) for i in range(12)] + [(1248, 2)])

    plsc.subcore_barrier()

    @pl.when(c == 0)
    def _c0out():
        pltpu.sync_copy(g_s.at[pl.ds(s * CP, CP)],
                        gout_l.at[pl.ds(s * CP, CP)])
        pltpu.sync_copy(deg_s.at[pl.ds(s * DEG_PT, DEG_PT)],
                        degout.at[pl.ds(s * DEG_PT, DEG_PT)])

        @pl.when(s == 0)
        def _crem0():
            pltpu.sync_copy(g_s.at[pl.ds(NS * CP, REM)],
                            gout_l.at[pl.ds(NS * CP, REM)])

    @pl.when(c == 1)
    def _c1out():
        pltpu.sync_copy(g_s.at[pl.ds(s * CP, CP)],
                        gout_r.at[pl.ds(s * CP, CP)])

        @pl.when(s == 0)
        def _crem1():
            pltpu.sync_copy(g_s.at[pl.ds(NS * CP, REM)],
                            gout_r.at[pl.ds(NS * CP, REM)])


def _lap_sc(src2b, dst2, w2, h2):
    mesh = plsc.VectorSubcoreMesh(core_axis_name="c", subcore_axis_name="s")
    fn = pl.kernel(
        _lap_body,
        out_type=[jax.ShapeDtypeStruct((N, HD), _f32),
                  jax.ShapeDtypeStruct((N, HD), _f32),
                  jax.ShapeDtypeStruct((DEG_PC,), _f32)],
        mesh=mesh,
        scratch_types=[
            pltpu.VMEM((8, 128), _i32),
            pltpu.VMEM((8, 128), _i32),
            pltpu.VMEM((8, 128), _f32),
            pltpu.VMEM((1024, HD), _f32),
            pltpu.VMEM((1024,), _f32),
            pltpu.VMEM_SHARED((N, HD), _f32),
            pltpu.VMEM_SHARED((DEG_PC,), _f32),
            pltpu.SemaphoreType.DMA,
            pltpu.SemaphoreType.DMA,
            pltpu.SemaphoreType.DMA,
            pltpu.SemaphoreType.DMA,
        ],
        compiler_params=_SC_PARAMS,
    )
    return fn(src2b, dst2, w2, h2)


BN1 = 2000
BE1 = 8000
BN2 = 2000


def _proj_body(x_ref, h_ref, wx_ref, wh_ref, ps_ref, pd_ref):
    acc = (jnp.dot(x_ref[:], wx_ref[:], preferred_element_type=_f32)
           + jnp.dot(h_ref[:], wh_ref[:], preferred_element_type=_f32))
    ps_ref[:] = acc[:, :DE]
    pd_ref[:] = acc[:, DE:]


def _q_body(ea_ref, he_ref, w1_ref, w2_ref, be_ref, q_ref):
    q_ref[:] = (jnp.dot(ea_ref[:], w1_ref[:], preferred_element_type=_f32)
                + jnp.dot(he_ref[:], w2_ref[:], preferred_element_type=_f32)
                + be_ref[:])


def _node_body(x_ref, hb_ref, aa_ref, ab_ref, gl_ref, gr_ref,
               da_ref, wx_ref, wh_ref, wa_ref, bn_ref,
               xn_ref, td_ref, sp_ref):
    agg = aa_ref[:] + ab_ref[:]
    xnew = jnp.maximum(
        jnp.dot(x_ref[:], wx_ref[:], preferred_element_type=_f32)
        + jnp.dot(hb_ref[:], wh_ref[:], preferred_element_type=_f32)
        + jnp.dot(agg, wa_ref[:], preferred_element_type=_f32)
        + bn_ref[:], 0.0)
    xn_ref[:] = xnew
    td_ref[:] = xnew - hb_ref[:]
    g = jnp.concatenate([gl_ref[:], gr_ref[:]], axis=1)
    sp_ref[:] = -COEFF * (da_ref[:] * hb_ref[:] - g)


def _proj_tc(x, h, wx, wh):
    return pl.pallas_call(
        _proj_body,
        grid=(N // BN1,),
        in_specs=[pl.BlockSpec((BN1, D), lambda i: (i, 0)),
                  pl.BlockSpec((BN1, D), lambda i: (i, 0)),
                  pl.BlockSpec((D, 2 * DE), lambda i: (0, 0)),
                  pl.BlockSpec((D, 2 * DE), lambda i: (0, 0))],
        out_specs=[pl.BlockSpec((BN1, DE), lambda i: (i, 0)),
                   pl.BlockSpec((BN1, DE), lambda i: (i, 0))],
        out_shape=[jax.ShapeDtypeStruct((N, DE), _f32),
                   jax.ShapeDtypeStruct((N, DE), _f32)],
    )(x, h, wx, wh)


EP8 = E // 8
BQ = 4000


def _q_tc(ea_r, he_r, w1bd, w2bd, be8):
    return pl.pallas_call(
        _q_body,
        grid=(EP8 // BQ,),
        in_specs=[pl.BlockSpec((BQ, 128), lambda i: (i, 0)),
                  pl.BlockSpec((BQ, 128), lambda i: (i, 0)),
                  pl.BlockSpec((128, 128), lambda i: (0, 0)),
                  pl.BlockSpec((128, 128), lambda i: (0, 0)),
                  pl.BlockSpec((1, 128), lambda i: (0, 0))],
        out_specs=pl.BlockSpec((BQ, 128), lambda i: (i, 0)),
        out_shape=jax.ShapeDtypeStruct((EP8, 128), _f32),
    )(ea_r, he_r, w1bd, w2bd, be8)


def _node_tc(x, h, aggout, gl, gr, deg2, wx, wh, wa, bn):
    return pl.pallas_call(
        _node_body,
        grid=(N // BN2,),
        in_specs=[
            pl.BlockSpec((BN2, D), lambda i: (i, 0)),
            pl.BlockSpec((BN2, D), lambda i: (i, 0)),
            pl.BlockSpec((BN2, DE), lambda i: (i, 0)),
            pl.BlockSpec((BN2, DE), lambda i: (i + N // BN2, 0)),
            pl.BlockSpec((BN2, HD), lambda i: (i, 0)),
            pl.BlockSpec((BN2, HD), lambda i: (i, 0)),
            pl.BlockSpec((BN2, 1), lambda i: (i, 0)),
            pl.BlockSpec((D, D), lambda i: (0, 0)),
            pl.BlockSpec((D, D), lambda i: (0, 0)),
            pl.BlockSpec((DE, D), lambda i: (0, 0)),
            pl.BlockSpec((1, D), lambda i: (0, 0)),
        ],
        out_specs=[pl.BlockSpec((BN2, D), lambda i: (i, 0)),
                   pl.BlockSpec((BN2, D), lambda i: (i, 0)),
                   pl.BlockSpec((BN2, D), lambda i: (i, 0))],
        out_shape=[jax.ShapeDtypeStruct((N, D), _f32),
                   jax.ShapeDtypeStruct((N, D), _f32),
                   jax.ShapeDtypeStruct((N, D), _f32)],
    )(x, h, aggout, aggout, gl, gr, deg2, wx, wh, wa, bn)


def kernel(x, edge_index, edge_attr, h_init_x, h_init_edge_attr,
           lap_weight, W_e, b_e, W_n, b_n):
    src2 = edge_index[0].reshape(IR, 128)
    dst2 = edge_index[1].reshape(IR, 128)
    w2 = lap_weight.reshape(IR, 128)
    src2b = jnp.concatenate([src2 * 2, src2 * 2 + 1], axis=0)
    h2 = h_init_x.reshape(2 * N, HD)

    gl, gr, degout = _lap_sc(src2b, dst2, w2, h2)

    wp_x = jnp.concatenate([W_e[32:160], W_e[288:416]], axis=1)
    wp_h = jnp.concatenate([W_e[160:288], W_e[416:544]], axis=1)
    psrc, pdst = _proj_tc(x, h_init_x, wp_x, wp_h)
    eye8 = jnp.eye(8, dtype=_f32)
    q2 = _q_tc(edge_attr.reshape(EP8, 128), h_init_edge_attr.reshape(EP8, 128),
               jnp.kron(eye8, W_e[:DE]), jnp.kron(eye8, W_e[DE:2 * DE]),
               jnp.tile(b_e, 8).reshape(1, 128))
    enew, aggout = _edge_sc(src2, dst2, q2, psrc, pdst)

    x_new, time_deriv, spatial_deriv = _node_tc(
        x, h_init_x, aggout, gl, gr, degout[:N].reshape(N, 1),
        W_n[:D], W_n[D:2 * D], W_n[2 * D:], b_n.reshape(1, D))

    return x_new, enew, time_deriv, spatial_deriv


# speedup vs baseline: 6.3621x; 1.1400x over previous
import jax
import jax.numpy as jnp
from jax import lax
from jax.experimental import pallas as pl
from jax.experimental.pallas import tpu as pltpu
from jax.experimental.pallas import tpu_sc as plsc

N = 10000
E = 160000
D = 128
DE = 16
HD = D // 2
COEFF = 0.1

NC = 2
NS = 16
NW = NC * NS
IR = E // 128

CP = 624
REM = N - NS * CP
DEG_PC = 16000
DEG_PT = DEG_PC // NS

_f32 = jnp.float32
_i32 = jnp.int32

_SC_PARAMS = pltpu.CompilerParams(use_tc_tiling_on_sc=False,
                                  needs_layout_passes=False)


def _edge_body(src2, dst2, q2, psrc, pdst, lap_done, enew, aggout,
               sidx, didx, qv, gs, gd, ev, agg_s, ps_s, pd_s, sem):
    del lap_done
    c = lax.axis_index("c")
    s = lax.axis_index("s")
    wid = s * NC + c

    pltpu.sync_copy(psrc.at[pl.ds(s * CP, CP)], ps_s.at[pl.ds(s * CP, CP)])
    pltpu.sync_copy(pdst.at[pl.ds(s * CP, CP)], pd_s.at[pl.ds(s * CP, CP)])

    def _zb(i, carry):
        ev[i, :] = jnp.zeros((16,), _f32)
        return carry
    lax.fori_loop(0, CP, _zb, 0, unroll=8)
    pltpu.sync_copy(ev.at[pl.ds(0, CP)], agg_s.at[pl.ds(s * CP, CP)])

    @pl.when(s == 0)
    def _zrem():
        pltpu.sync_copy(ev.at[pl.ds(0, REM)], agg_s.at[pl.ds(NS * CP, REM)])
        pltpu.sync_copy(psrc.at[pl.ds(NS * CP, REM)],
                        ps_s.at[pl.ds(NS * CP, REM)])
        pltpu.sync_copy(pdst.at[pl.ds(NS * CP, REM)],
                        pd_s.at[pl.ds(NS * CP, REM)])
    plsc.subcore_barrier()

    def do_chunk(rbase, nr):
        ebase = rbase * 128
        ne = nr * 128
        pltpu.sync_copy(src2.at[pl.ds(rbase, nr)], sidx.at[pl.ds(0, nr)])
        pltpu.sync_copy(dst2.at[pl.ds(rbase, nr)], didx.at[pl.ds(0, nr)])
        pltpu.sync_copy(q2.at[pl.ds(rbase * 16, nr * 16)],
                        qv.at[pl.ds(0, nr * 16)])
        descs = []
        for r in range(nr):
            descs.append(pltpu.async_copy(
                ps_s.at[sidx.at[r]], gs.at[pl.ds(r * 128, 128)], sem))
            descs.append(pltpu.async_copy(
                pd_s.at[didx.at[r]], gd.at[pl.ds(r * 128, 128)], sem))
        for dsc in descs:
            dsc.wait()

        def _cb(i, carry):
            ev[i, :] = jnp.maximum(
                qv[i // 8, pl.ds((i % 8) * 16, 16)] + gs[i, :] + gd[i, :],
                0.0)
            return carry
        lax.fori_loop(0, ne, _cb, 0, unroll=4)

        pltpu.sync_copy(ev.at[pl.ds(0, ne)], enew.at[pl.ds(ebase, ne)])
        for r in range(nr):
            pltpu.sync_copy(ev.at[pl.ds(r * 128, 128)],
                            agg_s.at[didx.at[r]], add=True)

    @pl.when(wid != NW - 1)
    def _main():
        for j in range(5):
            do_chunk(wid * 40 + j * 8, 8)

    @pl.when(wid == NW - 1)
    def _tail():
        do_chunk(1240, 8)
        do_chunk(1248, 2)

    plsc.subcore_barrier()
    pltpu.sync_copy(agg_s.at[pl.ds(s * CP, CP)],
                    aggout.at[pl.ds(c * N + s * CP, CP)])

    @pl.when(s == 0)
    def _crem():
        pltpu.sync_copy(agg_s.at[pl.ds(NS * CP, REM)],
                        aggout.at[pl.ds(c * N + NS * CP, REM)])


def _edge_sc(src2, dst2, q2, psrc, pdst, lap_done):
    mesh = plsc.VectorSubcoreMesh(core_axis_name="c", subcore_axis_name="s")
    fn = pl.kernel(
        _edge_body,
        out_type=[jax.ShapeDtypeStruct((E, DE), _f32),
                  jax.ShapeDtypeStruct((NC * N, DE), _f32)],
        mesh=mesh,
        scratch_types=[
            pltpu.VMEM((8, 128), _i32),
            pltpu.VMEM((8, 128), _i32),
            pltpu.VMEM((128, 128), _f32),
            pltpu.VMEM((1024, DE), _f32),
            pltpu.VMEM((1024, DE), _f32),
            pltpu.VMEM((1024, DE), _f32),
            pltpu.VMEM_SHARED((N, DE), _f32),
            pltpu.VMEM_SHARED((N, DE), _f32),
            pltpu.VMEM_SHARED((N, DE), _f32),
            pltpu.SemaphoreType.DMA,
        ],
        compiler_params=_SC_PARAMS,
    )
    return fn(src2, dst2, q2, psrc, pdst, lap_done)


def _lap_body(src2, dst2, w2, h2, gout_l, gout_r, degout,
              sidx, didx, wv, hrows, zv, g_s, deg_s,
              gsem0, gsem1, ssem0, ssem1):
    c = lax.axis_index("c")
    s = lax.axis_index("s")

    def _zb(i, carry):
        for cc in range(HD // 16):
            hrows[i, pl.ds(cc * 16, 16)] = jnp.zeros((16,), _f32)
        return carry
    lax.fori_loop(0, 1024, _zb, 0, unroll=4)

    def _zvb(i, carry):
        zv[pl.ds(i * 16, 16)] = jnp.zeros((16,), _f32)
        return carry
    lax.fori_loop(0, 64, _zvb, 0, unroll=8)

    pltpu.sync_copy(hrows.at[pl.ds(0, CP)], g_s.at[pl.ds(s * CP, CP)])

    @pl.when(s == 0)
    def _zrem():
        pltpu.sync_copy(hrows.at[pl.ds(0, REM)], g_s.at[pl.ds(NS * CP, REM)])
    pltpu.sync_copy(zv.at[pl.ds(0, DEG_PT)],
                    deg_s.at[pl.ds(s * DEG_PT, DEG_PT)])
    plsc.subcore_barrier()

    gsems = (gsem0, gsem1)
    ssems = (ssem0, ssem1)

    def run(chunks):
        pend_g = [[], []]
        pend_s = [[], []]

        def load_chunk(i):
            rb, nr = chunks[i]
            b = i % 2
            r0 = 4 * b
            pltpu.sync_copy(src2.at[pl.ds(rb, nr)], sidx.at[pl.ds(r0, nr)])
            pltpu.sync_copy(dst2.at[pl.ds(rb, nr)], didx.at[pl.ds(r0, nr)])
            pltpu.sync_copy(w2.at[pl.ds(rb, nr)], wv.at[pl.ds(r0, nr)])
            for r in range(nr):
                for k in range(8):
                    v = sidx[r0 + r, pl.ds(k * 16, 16)]
                    sidx[r0 + r, pl.ds(k * 16, 16)] = v + v + c
            for r in range(nr):
                pend_g[b].append(pltpu.async_copy(
                    h2.at[sidx.at[r0 + r]],
                    hrows.at[pl.ds((r0 + r) * 128, 128)], gsems[b]))

        load_chunk(0)
        for i in range(len(chunks)):
            rb, nr = chunks[i]
            b = i % 2
            r0 = 4 * b
            if i + 1 < len(chunks):
                for dsc in pend_s[1 - b]:
                    dsc.wait()
                pend_s[1 - b] = []
                load_chunk(i + 1)
            for dsc in pend_g[b]:
                dsc.wait()
            pend_g[b] = []

            def _sb(e, carry):
                wspl = plsc.load_gather(
                    wv, [jnp.full((16,), r0 + e // 128, _i32),
                         jnp.full((16,), e % 128, _i32)])
                for cc in range(HD // 16):
                    hrows[r0 * 128 + e, pl.ds(cc * 16, 16)] = (
                        hrows[r0 * 128 + e, pl.ds(cc * 16, 16)] * wspl)
                return carry
            lax.fori_loop(0, nr * 128, _sb, 0)

            for r in range(nr):
                pend_s[b].append(pltpu.async_copy(
                    hrows.at[pl.ds((r0 + r) * 128, 128)],
                    g_s.at[didx.at[r0 + r]], ssems[b], add=True))
                pend_s[b].append(pltpu.async_copy(
                    wv.at[r0 + r], deg_s.at[didx.at[r0 + r]], ssems[b],
                    add=True))
        for b in range(2):
            for dsc in pend_s[b]:
                dsc.wait()

    @pl.when(s != NS - 1)
    def _main():
        run([(s * 80 + i * 4, 4) for i in range(20)])

    @pl.when(s == NS - 1)
    def _tail():
        run([(1200 + i * 4, 4) for i in range(12)] + [(1248, 2)])

    plsc.subcore_barrier()

    @pl.when(c == 0)
    def _c0out():
        pltpu.sync_copy(g_s.at[pl.ds(s * CP, CP)],
                        gout_l.at[pl.ds(s * CP, CP)])
        pltpu.sync_copy(deg_s.at[pl.ds(s * DEG_PT, DEG_PT)],
                        degout.at[pl.ds(s * DEG_PT, DEG_PT)])

        @pl.when(s == 0)
        def _crem0():
            pltpu.sync_copy(g_s.at[pl.ds(NS * CP, REM)],
                            gout_l.at[pl.ds(NS * CP, REM)])

    @pl.when(c == 1)
    def _c1out():
        pltpu.sync_copy(g_s.at[pl.ds(s * CP, CP)],
                        gout_r.at[pl.ds(s * CP, CP)])

        @pl.when(s == 0)
        def _crem1():
            pltpu.sync_copy(g_s.at[pl.ds(NS * CP, REM)],
                            gout_r.at[pl.ds(NS * CP, REM)])


def _lap_sc(src2, dst2, w2, h2):
    mesh = plsc.VectorSubcoreMesh(core_axis_name="c", subcore_axis_name="s")
    fn = pl.kernel(
        _lap_body,
        out_type=[jax.ShapeDtypeStruct((N, HD), _f32),
                  jax.ShapeDtypeStruct((N, HD), _f32),
                  jax.ShapeDtypeStruct((DEG_PC,), _f32)],
        mesh=mesh,
        scratch_types=[
            pltpu.VMEM((8, 128), _i32),
            pltpu.VMEM((8, 128), _i32),
            pltpu.VMEM((8, 128), _f32),
            pltpu.VMEM((1024, HD), _f32),
            pltpu.VMEM((1024,), _f32),
            pltpu.VMEM_SHARED((N, HD), _f32),
            pltpu.VMEM_SHARED((DEG_PC,), _f32),
            pltpu.SemaphoreType.DMA,
            pltpu.SemaphoreType.DMA,
            pltpu.SemaphoreType.DMA,
            pltpu.SemaphoreType.DMA,
        ],
        compiler_params=_SC_PARAMS,
    )
    return fn(src2, dst2, w2, h2)


BN1 = 2000
BE1 = 8000
BN2 = 2000


def _proj_body(x_ref, h_ref, wx_ref, wh_ref, ps_ref, pd_ref):
    acc = (jnp.dot(x_ref[:], wx_ref[:], preferred_element_type=_f32)
           + jnp.dot(h_ref[:], wh_ref[:], preferred_element_type=_f32))
    ps_ref[:] = acc[:, :DE]
    pd_ref[:] = acc[:, DE:]


def _q_body(ea_ref, he_ref, w1_ref, w2_ref, be_ref, q_ref):
    q_ref[:] = (jnp.dot(ea_ref[:], w1_ref[:], preferred_element_type=_f32)
                + jnp.dot(he_ref[:], w2_ref[:], preferred_element_type=_f32)
                + be_ref[:])


def _node_body(x_ref, hb_ref, aa_ref, ab_ref, gl_ref, gr_ref,
               da_ref, wx_ref, wh_ref, wa_ref, bn_ref,
               xn_ref, td_ref, sp_ref):
    agg = aa_ref[:] + ab_ref[:]
    xnew = jnp.maximum(
        jnp.dot(x_ref[:], wx_ref[:], preferred_element_type=_f32)
        + jnp.dot(hb_ref[:], wh_ref[:], preferred_element_type=_f32)
        + jnp.dot(agg, wa_ref[:], preferred_element_type=_f32)
        + bn_ref[:], 0.0)
    xn_ref[:] = xnew
    td_ref[:] = xnew - hb_ref[:]
    g = jnp.concatenate([gl_ref[:], gr_ref[:]], axis=1)
    sp_ref[:] = -COEFF * (da_ref[:] * hb_ref[:] - g)


def _proj_tc(x, h, wx, wh):
    return pl.pallas_call(
        _proj_body,
        grid=(N // BN1,),
        in_specs=[pl.BlockSpec((BN1, D), lambda i: (i, 0)),
                  pl.BlockSpec((BN1, D), lambda i: (i, 0)),
                  pl.BlockSpec((D, 2 * DE), lambda i: (0, 0)),
                  pl.BlockSpec((D, 2 * DE), lambda i: (0, 0))],
        out_specs=[pl.BlockSpec((BN1, DE), lambda i: (i, 0)),
                   pl.BlockSpec((BN1, DE), lambda i: (i, 0))],
        out_shape=[jax.ShapeDtypeStruct((N, DE), _f32),
                   jax.ShapeDtypeStruct((N, DE), _f32)],
    )(x, h, wx, wh)


EP8 = E // 8
BQ = 4000


def _q_tc(ea_r, he_r, w1bd, w2bd, be8):
    return pl.pallas_call(
        _q_body,
        grid=(EP8 // BQ,),
        in_specs=[pl.BlockSpec((BQ, 128), lambda i: (i, 0)),
                  pl.BlockSpec((BQ, 128), lambda i: (i, 0)),
                  pl.BlockSpec((128, 128), lambda i: (0, 0)),
                  pl.BlockSpec((128, 128), lambda i: (0, 0)),
                  pl.BlockSpec((1, 128), lambda i: (0, 0))],
        out_specs=pl.BlockSpec((BQ, 128), lambda i: (i, 0)),
        out_shape=jax.ShapeDtypeStruct((EP8, 128), _f32),
    )(ea_r, he_r, w1bd, w2bd, be8)


def _node_tc(x, h, aggout, gl, gr, deg2, wx, wh, wa, bn):
    return pl.pallas_call(
        _node_body,
        grid=(N // BN2,),
        in_specs=[
            pl.BlockSpec((BN2, D), lambda i: (i, 0)),
            pl.BlockSpec((BN2, D), lambda i: (i, 0)),
            pl.BlockSpec((BN2, DE), lambda i: (i, 0)),
            pl.BlockSpec((BN2, DE), lambda i: (i + N // BN2, 0)),
            pl.BlockSpec((BN2, HD), lambda i: (i, 0)),
            pl.BlockSpec((BN2, HD), lambda i: (i, 0)),
            pl.BlockSpec((BN2, 1), lambda i: (i, 0)),
            pl.BlockSpec((D, D), lambda i: (0, 0)),
            pl.BlockSpec((D, D), lambda i: (0, 0)),
            pl.BlockSpec((DE, D), lambda i: (0, 0)),
            pl.BlockSpec((1, D), lambda i: (0, 0)),
        ],
        out_specs=[pl.BlockSpec((BN2, D), lambda i: (i, 0)),
                   pl.BlockSpec((BN2, D), lambda i: (i, 0)),
                   pl.BlockSpec((BN2, D), lambda i: (i, 0))],
        out_shape=[jax.ShapeDtypeStruct((N, D), _f32),
                   jax.ShapeDtypeStruct((N, D), _f32),
                   jax.ShapeDtypeStruct((N, D), _f32)],
    )(x, h, aggout, aggout, gl, gr, deg2, wx, wh, wa, bn)


def kernel(x, edge_index, edge_attr, h_init_x, h_init_edge_attr,
           lap_weight, W_e, b_e, W_n, b_n):
    src2 = edge_index[0].reshape(IR, 128)
    dst2 = edge_index[1].reshape(IR, 128)
    w2 = lap_weight.reshape(IR, 128)
    h2 = h_init_x.reshape(2 * N, HD)

    gl, gr, degout = _lap_sc(src2, dst2, w2, h2)

    wp_x = jnp.concatenate([W_e[32:160], W_e[288:416]], axis=1)
    wp_h = jnp.concatenate([W_e[160:288], W_e[416:544]], axis=1)
    psrc, pdst = _proj_tc(x, h_init_x, wp_x, wp_h)
    eye8 = jnp.eye(8, dtype=_f32)
    q2 = _q_tc(edge_attr.reshape(EP8, 128), h_init_edge_attr.reshape(EP8, 128),
               jnp.kron(eye8, W_e[:DE]), jnp.kron(eye8, W_e[DE:2 * DE]),
               jnp.tile(b_e, 8).reshape(1, 128))
    enew, aggout = _edge_sc(src2, dst2, q2, psrc, pdst, degout)

    x_new, time_deriv, spatial_deriv = _node_tc(
        x, h_init_x, aggout, gl, gr, degout[:N].reshape(N, 1),
        W_n[:D], W_n[D:2 * D], W_n[2 * D:], b_n.reshape(1, D))

    return x_new, enew, time_deriv, spatial_deriv


# speedup vs baseline: 6.3762x; 1.0022x over previous
import jax
import jax.numpy as jnp
from jax import lax
from jax.experimental import pallas as pl
from jax.experimental.pallas import tpu as pltpu
from jax.experimental.pallas import tpu_sc as plsc

N = 10000
E = 160000
D = 128
DE = 16
HD = D // 2
COEFF = 0.1

NC = 2
NS = 16
NW = NC * NS
IR = E // 128

CP = 624
REM = N - NS * CP
DEG_PC = 16000
DEG_PT = DEG_PC // NS

_f32 = jnp.float32
_i32 = jnp.int32

_SC_PARAMS = pltpu.CompilerParams(use_tc_tiling_on_sc=False,
                                  needs_layout_passes=False)


def _edge_body(src2, dst2, q2, psrc, pdst, lap_done, enew, aggout,
               sidx, didx, qv, gs, gd, ev, agg_s, ps_s, pd_s, sem):
    del lap_done
    c = lax.axis_index("c")
    s = lax.axis_index("s")
    wid = s * NC + c

    pltpu.sync_copy(psrc.at[pl.ds(s * CP, CP)], ps_s.at[pl.ds(s * CP, CP)])
    pltpu.sync_copy(pdst.at[pl.ds(s * CP, CP)], pd_s.at[pl.ds(s * CP, CP)])

    def _zb(i, carry):
        ev[i, :] = jnp.zeros((16,), _f32)
        return carry
    lax.fori_loop(0, CP, _zb, 0, unroll=8)
    pltpu.sync_copy(ev.at[pl.ds(0, CP)], agg_s.at[pl.ds(s * CP, CP)])

    @pl.when(s == 0)
    def _zrem():
        pltpu.sync_copy(ev.at[pl.ds(0, REM)], agg_s.at[pl.ds(NS * CP, REM)])
        pltpu.sync_copy(psrc.at[pl.ds(NS * CP, REM)],
                        ps_s.at[pl.ds(NS * CP, REM)])
        pltpu.sync_copy(pdst.at[pl.ds(NS * CP, REM)],
                        pd_s.at[pl.ds(NS * CP, REM)])
    plsc.subcore_barrier()

    def do_chunk(rbase, nr):
        ebase = rbase * 128
        ne = nr * 128
        pltpu.sync_copy(src2.at[pl.ds(rbase, nr)], sidx.at[pl.ds(0, nr)])
        pltpu.sync_copy(dst2.at[pl.ds(rbase, nr)], didx.at[pl.ds(0, nr)])
        pltpu.sync_copy(q2.at[pl.ds(rbase * 16, nr * 16)],
                        qv.at[pl.ds(0, nr * 16)])
        descs = []
        for r in range(nr):
            descs.append(pltpu.async_copy(
                ps_s.at[sidx.at[r]], gs.at[pl.ds(r * 128, 128)], sem))
            descs.append(pltpu.async_copy(
                pd_s.at[didx.at[r]], gd.at[pl.ds(r * 128, 128)], sem))
        for dsc in descs:
            dsc.wait()

        def _cb(i, carry):
            ev[i, :] = jnp.maximum(
                qv[i // 8, pl.ds((i % 8) * 16, 16)] + gs[i, :] + gd[i, :],
                0.0)
            return carry
        lax.fori_loop(0, ne, _cb, 0, unroll=4)

        pltpu.sync_copy(ev.at[pl.ds(0, ne)], enew.at[pl.ds(ebase, ne)])
        for r in range(nr):
            pltpu.sync_copy(ev.at[pl.ds(r * 128, 128)],
                            agg_s.at[didx.at[r]], add=True)

    @pl.when(wid != NW - 1)
    def _main():
        for j in range(5):
            do_chunk(wid * 40 + j * 8, 8)

    @pl.when(wid == NW - 1)
    def _tail():
        do_chunk(1240, 8)
        do_chunk(1248, 2)

    plsc.subcore_barrier()
    pltpu.sync_copy(agg_s.at[pl.ds(s * CP, CP)],
                    aggout.at[pl.ds(c * N + s * CP, CP)])

    @pl.when(s == 0)
    def _crem():
        pltpu.sync_copy(agg_s.at[pl.ds(NS * CP, REM)],
                        aggout.at[pl.ds(c * N + NS * CP, REM)])


def _edge_sc(src2, dst2, q2, psrc, pdst, lap_done):
    mesh = plsc.VectorSubcoreMesh(core_axis_name="c", subcore_axis_name="s")
    fn = pl.kernel(
        _edge_body,
        out_type=[jax.ShapeDtypeStruct((E, DE), _f32),
                  jax.ShapeDtypeStruct((NC * N, DE), _f32)],
        mesh=mesh,
        scratch_types=[
            pltpu.VMEM((8, 128), _i32),
            pltpu.VMEM((8, 128), _i32),
            pltpu.VMEM((128, 128), _f32),
            pltpu.VMEM((1024, DE), _f32),
            pltpu.VMEM((1024, DE), _f32),
            pltpu.VMEM((1024, DE), _f32),
            pltpu.VMEM_SHARED((N, DE), _f32),
            pltpu.VMEM_SHARED((N, DE), _f32),
            pltpu.VMEM_SHARED((N, DE), _f32),
            pltpu.SemaphoreType.DMA,
        ],
        compiler_params=_SC_PARAMS,
    )
    return fn(src2, dst2, q2, psrc, pdst, lap_done)


def _lap_body(src2, dst2, w2, h2, gout_l, gout_r, degout,
              sidx, didx, wv, hrows, zv, g_s, deg_s,
              gsem0, gsem1, ssem0, ssem1):
    c = lax.axis_index("c")
    s = lax.axis_index("s")

    def _zb(i, carry):
        for cc in range(HD // 16):
            hrows[i, pl.ds(cc * 16, 16)] = jnp.zeros((16,), _f32)
        return carry
    lax.fori_loop(0, 1024, _zb, 0, unroll=4)

    def _zvb(i, carry):
        zv[pl.ds(i * 16, 16)] = jnp.zeros((16,), _f32)
        return carry
    lax.fori_loop(0, 64, _zvb, 0, unroll=8)

    pltpu.sync_copy(hrows.at[pl.ds(0, CP)], g_s.at[pl.ds(s * CP, CP)])

    @pl.when(s == 0)
    def _zrem():
        pltpu.sync_copy(hrows.at[pl.ds(0, REM)], g_s.at[pl.ds(NS * CP, REM)])
    pltpu.sync_copy(zv.at[pl.ds(0, DEG_PT)],
                    deg_s.at[pl.ds(s * DEG_PT, DEG_PT)])
    plsc.subcore_barrier()

    gsems = (gsem0, gsem1)
    ssems = (ssem0, ssem1)

    def run(chunks):
        pend_g = [[], []]
        pend_s = [[], []]

        def load_chunk(i):
            rb, nr = chunks[i]
            b = i % 2
            r0 = 4 * b
            pltpu.sync_copy(src2.at[pl.ds(rb, nr)], sidx.at[pl.ds(r0, nr)])
            pltpu.sync_copy(dst2.at[pl.ds(rb, nr)], didx.at[pl.ds(r0, nr)])
            pltpu.sync_copy(w2.at[pl.ds(rb, nr)], wv.at[pl.ds(r0, nr)])
            for r in range(nr):
                for k in range(8):
                    v = sidx[r0 + r, pl.ds(k * 16, 16)]
                    sidx[r0 + r, pl.ds(k * 16, 16)] = v + v + c
            for r in range(nr):
                pend_g[b].append(pltpu.async_copy(
                    h2.at[sidx.at[r0 + r]],
                    hrows.at[pl.ds((r0 + r) * 128, 128)], gsems[b]))

        load_chunk(0)
        for i in range(len(chunks)):
            rb, nr = chunks[i]
            b = i % 2
            r0 = 4 * b
            if i + 1 < len(chunks):
                for dsc in pend_s[1 - b]:
                    dsc.wait()
                pend_s[1 - b] = []
                load_chunk(i + 1)
            for dsc in pend_g[b]:
                dsc.wait()
            pend_g[b] = []

            def _sb(e, carry):
                wspl = plsc.load_gather(
                    wv, [jnp.full((16,), r0 + e // 128, _i32),
                         jnp.full((16,), e % 128, _i32)])
                for cc in range(HD // 16):
                    hrows[r0 * 128 + e, pl.ds(cc * 16, 16)] = (
                        hrows[r0 * 128 + e, pl.ds(cc * 16, 16)] * wspl)
                return carry
            lax.fori_loop(0, nr * 128, _sb, 0, unroll=4)

            for r in range(nr):
                pend_s[b].append(pltpu.async_copy(
                    hrows.at[pl.ds((r0 + r) * 128, 128)],
                    g_s.at[didx.at[r0 + r]], ssems[b], add=True))
                pend_s[b].append(pltpu.async_copy(
                    wv.at[r0 + r], deg_s.at[didx.at[r0 + r]], ssems[b],
                    add=True))
        for b in range(2):
            for dsc in pend_s[b]:
                dsc.wait()

    @pl.when(s != NS - 1)
    def _main():
        run([(s * 80 + i * 4, 4) for i in range(20)])

    @pl.when(s == NS - 1)
    def _tail():
        run([(1200 + i * 4, 4) for i in range(12)] + [(1248, 2)])

    plsc.subcore_barrier()

    @pl.when(c == 0)
    def _c0out():
        pltpu.sync_copy(g_s.at[pl.ds(s * CP, CP)],
                        gout_l.at[pl.ds(s * CP, CP)])
        pltpu.sync_copy(deg_s.at[pl.ds(s * DEG_PT, DEG_PT)],
                        degout.at[pl.ds(s * DEG_PT, DEG_PT)])

        @pl.when(s == 0)
        def _crem0():
            pltpu.sync_copy(g_s.at[pl.ds(NS * CP, REM)],
                            gout_l.at[pl.ds(NS * CP, REM)])

    @pl.when(c == 1)
    def _c1out():
        pltpu.sync_copy(g_s.at[pl.ds(s * CP, CP)],
                        gout_r.at[pl.ds(s * CP, CP)])

        @pl.when(s == 0)
        def _crem1():
            pltpu.sync_copy(g_s.at[pl.ds(NS * CP, REM)],
                            gout_r.at[pl.ds(NS * CP, REM)])


def _lap_sc(src2, dst2, w2, h2):
    mesh = plsc.VectorSubcoreMesh(core_axis_name="c", subcore_axis_name="s")
    fn = pl.kernel(
        _lap_body,
        out_type=[jax.ShapeDtypeStruct((N, HD), _f32),
                  jax.ShapeDtypeStruct((N, HD), _f32),
                  jax.ShapeDtypeStruct((DEG_PC,), _f32)],
        mesh=mesh,
        scratch_types=[
            pltpu.VMEM((8, 128), _i32),
            pltpu.VMEM((8, 128), _i32),
            pltpu.VMEM((8, 128), _f32),
            pltpu.VMEM((1024, HD), _f32),
            pltpu.VMEM((1024,), _f32),
            pltpu.VMEM_SHARED((N, HD), _f32),
            pltpu.VMEM_SHARED((DEG_PC,), _f32),
            pltpu.SemaphoreType.DMA,
            pltpu.SemaphoreType.DMA,
            pltpu.SemaphoreType.DMA,
            pltpu.SemaphoreType.DMA,
        ],
        compiler_params=_SC_PARAMS,
    )
    return fn(src2, dst2, w2, h2)


BN1 = 2000
BE1 = 8000
BN2 = 2000


def _proj_body(x_ref, h_ref, wx_ref, wh_ref, ps_ref, pd_ref):
    acc = (jnp.dot(x_ref[:], wx_ref[:], preferred_element_type=_f32)
           + jnp.dot(h_ref[:], wh_ref[:], preferred_element_type=_f32))
    ps_ref[:] = acc[:, :DE]
    pd_ref[:] = acc[:, DE:]


def _q_body(ea_ref, he_ref, w1_ref, w2_ref, be_ref, q_ref):
    q_ref[:] = (jnp.dot(ea_ref[:], w1_ref[:], preferred_element_type=_f32)
                + jnp.dot(he_ref[:], w2_ref[:], preferred_element_type=_f32)
                + be_ref[:])


def _node_body(x_ref, hb_ref, aa_ref, ab_ref, gl_ref, gr_ref,
               da_ref, wx_ref, wh_ref, wa_ref, bn_ref,
               xn_ref, td_ref, sp_ref):
    agg = aa_ref[:] + ab_ref[:]
    agg_con = jnp.dot(agg, wa_ref[:], preferred_element_type=_f32)
    xnew = jnp.maximum(
        jnp.dot(x_ref[:], wx_ref[:], preferred_element_type=_f32)
        + jnp.dot(hb_ref[:], wh_ref[:], preferred_element_type=_f32)
        + agg_con
        + bn_ref[:], 0.0)
    xn_ref[:] = xnew
    td_ref[:] = xnew - hb_ref[:]
    g = jnp.concatenate([gl_ref[:], gr_ref[:]], axis=1)
    sp_ref[:] = -COEFF * (da_ref[:] * hb_ref[:] - g)


def _proj_tc(x, h, wx, wh):
    return pl.pallas_call(
        _proj_body,
        grid=(N // BN1,),
        in_specs=[pl.BlockSpec((BN1, D), lambda i: (i, 0)),
                  pl.BlockSpec((BN1, D), lambda i: (i, 0)),
                  pl.BlockSpec((D, 2 * DE), lambda i: (0, 0)),
                  pl.BlockSpec((D, 2 * DE), lambda i: (0, 0))],
        out_specs=[pl.BlockSpec((BN1, DE), lambda i: (i, 0)),
                   pl.BlockSpec((BN1, DE), lambda i: (i, 0))],
        out_shape=[jax.ShapeDtypeStruct((N, DE), _f32),
                   jax.ShapeDtypeStruct((N, DE), _f32)],
    )(x, h, wx, wh)


EP8 = E // 8
BQ = 4000


def _q_tc(ea_r, he_r, w1bd, w2bd, be8):
    return pl.pallas_call(
        _q_body,
        grid=(EP8 // BQ,),
        in_specs=[pl.BlockSpec((BQ, 128), lambda i: (i, 0)),
                  pl.BlockSpec((BQ, 128), lambda i: (i, 0)),
                  pl.BlockSpec((128, 128), lambda i: (0, 0)),
                  pl.BlockSpec((128, 128), lambda i: (0, 0)),
                  pl.BlockSpec((1, 128), lambda i: (0, 0))],
        out_specs=pl.BlockSpec((BQ, 128), lambda i: (i, 0)),
        out_shape=jax.ShapeDtypeStruct((EP8, 128), _f32),
    )(ea_r, he_r, w1bd, w2bd, be8)


def _node_tc(x, h, aggout, gl, gr, deg2, wx, wh, wa, bn):
    return pl.pallas_call(
        _node_body,
        grid=(1,),
        in_specs=[
            pl.BlockSpec((N, D), lambda i: (0, 0)),
            pl.BlockSpec((N, D), lambda i: (0, 0)),
            pl.BlockSpec((N, DE), lambda i: (0, 0)),
            pl.BlockSpec((N, DE), lambda i: (1, 0)),
            pl.BlockSpec((N, HD), lambda i: (0, 0)),
            pl.BlockSpec((N, HD), lambda i: (0, 0)),
            pl.BlockSpec((N, 1), lambda i: (0, 0)),
            pl.BlockSpec((D, D), lambda i: (0, 0)),
            pl.BlockSpec((D, D), lambda i: (0, 0)),
            pl.BlockSpec((DE, D), lambda i: (0, 0)),
            pl.BlockSpec((1, D), lambda i: (0, 0)),
        ],
        out_specs=[pl.BlockSpec((N, D), lambda i: (0, 0)),
                   pl.BlockSpec((N, D), lambda i: (0, 0)),
                   pl.BlockSpec((N, D), lambda i: (0, 0))],
        out_shape=[jax.ShapeDtypeStruct((N, D), _f32),
                   jax.ShapeDtypeStruct((N, D), _f32),
                   jax.ShapeDtypeStruct((N, D), _f32)],
    )(x, h, aggout, aggout, gl, gr, deg2, wx, wh, wa, bn)


def kernel(x, edge_index, edge_attr, h_init_x, h_init_edge_attr,
           lap_weight, W_e, b_e, W_n, b_n):
    src2 = edge_index[0].reshape(IR, 128)
    dst2 = edge_index[1].reshape(IR, 128)
    w2 = lap_weight.reshape(IR, 128)
    h2 = h_init_x.reshape(2 * N, HD)

    gl, gr, degout = _lap_sc(src2, dst2, w2, h2)

    wp_x = jnp.concatenate([W_e[32:160], W_e[288:416]], axis=1)
    wp_h = jnp.concatenate([W_e[160:288], W_e[416:544]], axis=1)
    psrc, pdst = _proj_tc(x, h_init_x, wp_x, wp_h)
    eye8 = jnp.eye(8, dtype=_f32)
    q2 = _q_tc(edge_attr.reshape(EP8, 128), h_init_edge_attr.reshape(EP8, 128),
               jnp.kron(eye8, W_e[:DE]), jnp.kron(eye8, W_e[DE:2 * DE]),
               jnp.tile(b_e, 8).reshape(1, 128))
    enew, aggout = _edge_sc(src2, dst2, q2, psrc, pdst, degout)

    x_new, time_deriv, spatial_deriv = _node_tc(
        x, h_init_x, aggout, gl, gr, degout[:N].reshape(N, 1),
        W_n[:D], W_n[D:2 * D], W_n[2 * D:], b_n.reshape(1, D))

    return x_new, enew, time_deriv, spatial_deriv
